# baseline (speedup 1.0000x reference)
import functools

import jax
import jax.numpy as jnp
from jax import lax
from jax.experimental import pallas as pl
from jax.experimental.pallas import tpu as pltpu
from jax.experimental.pallas import tpu_sc as plsc

B, N, DIM, C, K = 32, 49, 128, 32, 128
TAU, TAU_STU = 0.04, 0.1
R = C * K
ROW = 2 * N * DIM
BK = 8
NW_GATHER = 8


def _argmax_first(x, size):
    m = jnp.max(x, axis=1, keepdims=True)
    ii = lax.broadcasted_iota(jnp.int32, x.shape, 1)
    return jnp.min(jnp.where(x == m, ii, size), axis=1, keepdims=True)


def _route_body(c_ref, ptr_ref, g2_ref, u_ref, gB_ref,
                r2_ref, val_ref, ridx_ref, nptr_ref):
    cpt = c_ref[...]
    ptrf = ptr_ref[...]

    cpt2 = jnp.concatenate([cpt, cpt], axis=0)
    mask2 = jnp.sum(cpt2, axis=1, keepdims=True) == 0
    logits2 = jnp.log(cpt2 + mask2.astype(jnp.float32) + 1e-12) + g2_ref[...]
    cs2 = _argmax_first(logits2, C)
    oh2 = (cs2 == lax.broadcasted_iota(jnp.int32, (2 * B, C), 1)).astype(jnp.float32)
    p2 = jnp.dot(oh2, ptrf, preferred_element_type=jnp.float32)
    size2 = jnp.clip(p2[:, 1:2], 0.0, float(K))
    pos2 = jnp.minimum(
        jnp.floor(u_ref[...] * jnp.maximum(size2, 1.0)).astype(jnp.int32), K - 1)
    valid = (size2 != 0.0) & jnp.logical_not(mask2)
    r2_ref[...] = cs2 * K + pos2
    val_ref[...] = valid.astype(jnp.int32)

    maskB = jnp.sum(cpt, axis=1, keepdims=True) == 0
    logitsB = jnp.log(cpt + maskB.astype(jnp.float32) + 1e-12) + gB_ref[...]
    csB = _argmax_first(logitsB, C)
    ohB = (csB == lax.broadcasted_iota(jnp.int32, (B, C), 1)).astype(jnp.float32)
    pB = jnp.dot(ohB, ptrf, preferred_element_type=jnp.float32)
    ptrB0 = pB[:, 0:1].astype(jnp.int32)
    ptrB1f = pB[:, 1:2]
    posB = lax.rem(ptrB0 + 1, K)
    do = jnp.logical_not(maskB)
    rB = csB * K + posB

    S = (lax.broadcasted_iota(jnp.int32, (B, B), 0)
         < lax.broadcasted_iota(jnp.int32, (B, B), 1)).astype(jnp.float32)
    ohR = (rB == lax.broadcasted_iota(jnp.int32, (B, R), 1)).astype(jnp.float32)
    later = jnp.dot(S, ohR, preferred_element_type=jnp.float32)
    eff = do & (jnp.sum(ohR * later, axis=1, keepdims=True) == 0)
    laterB = jnp.dot(S, ohB, preferred_element_type=jnp.float32)
    effp = do & (jnp.sum(ohB * laterB, axis=1, keepdims=True) == 0)
    ridx_ref[...] = jnp.where(eff, rB, -1)

    w = ohB * effp.astype(jnp.float32)
    dn = (((0,), (0,)), ((), ()))
    hit = lax.dot_general(w, jnp.ones((B, 1), jnp.float32), dn,
                          preferred_element_type=jnp.float32)
    n0 = lax.dot_general(w, posB.astype(jnp.float32), dn,
                         preferred_element_type=jnp.float32)
    v1 = jnp.clip(ptrB1f + 1.0, 0.0, float(K))
    n1 = lax.dot_general(w, v1, dn, preferred_element_type=jnp.float32)
    new0 = jnp.where(hit > 0, n0, ptrf[:, 0:1])
    new1 = jnp.where(hit > 0, n1, ptrf[:, 1:2])
    nptr_ref[...] = jnp.concatenate([new0, new1], axis=1).astype(jnp.int32)


def _route(concept, queue_ptr, g2, u, gB):
    return pl.pallas_call(
        _route_body,
        out_shape=[
            jax.ShapeDtypeStruct((2 * B, 1), jnp.int32),
            jax.ShapeDtypeStruct((2 * B, 1), jnp.int32),
            jax.ShapeDtypeStruct((B, 1), jnp.int32),
            jax.ShapeDtypeStruct((C, 2), jnp.int32),
        ],
    )(concept, queue_ptr.astype(jnp.float32), g2, u[:, None], gB)


def _sc_gather(qg_flat, idx2):
    info = plsc.get_sparse_core_info()
    rows_per_w = (2 * B) // NW_GATHER

    @functools.partial(
        pl.kernel,
        mesh=plsc.VectorSubcoreMesh(core_axis_name="c", subcore_axis_name="s"),
        out_type=jax.ShapeDtypeStruct((2 * B, ROW), jnp.float32),
        scratch_types=[
            pltpu.VMEM((rows_per_w,), jnp.int32),
            pltpu.VMEM((rows_per_w, ROW), jnp.float32),
            pltpu.SemaphoreType.DMA,
        ],
    )
    def gather(qg_ref, idx_ref, out_ref, idx_v, rows_v, sem):
        wid = lax.axis_index("s") * info.num_cores + lax.axis_index("c")

        @pl.when(wid < NW_GATHER)
        def _():
            pltpu.sync_copy(idx_ref.at[wid], idx_v)
            pltpu.async_copy(qg_ref.at[idx_v], rows_v, sem).wait()
            pltpu.sync_copy(rows_v, out_ref.at[pl.ds(wid * rows_per_w, rows_per_w)])

    return gather(qg_flat, idx2)


def _bigcopy_body(ridx_ref, qg_ref, q_ref, gs_ref, tcls_ref, og_ref, oq_ref):
    og_ref[...] = qg_ref[...]
    oq_ref[...] = q_ref[...]
    base = pl.program_id(0) * BK
    for i in range(B):
        r = ridx_ref[i]

        @pl.when((r >= base) & (r < base + BK))
        def _(i=i, r=r):
            k = r - base
            og_ref[pl.ds(k, 1)] = gs_ref[i][None]
            oq_ref[pl.ds(k, 1)] = tcls_ref[i][None]


def _bigcopy(ridx, qg3, q2d, grid_src, tclsB):
    grid_spec = pltpu.PrefetchScalarGridSpec(
        num_scalar_prefetch=1,
        grid=(R // BK,),
        in_specs=[
            pl.BlockSpec((BK, 2 * N, DIM), lambda g, s: (g, 0, 0)),
            pl.BlockSpec((BK, DIM), lambda g, s: (g, 0)),
            pl.BlockSpec((B, 2 * N, DIM), lambda g, s: (0, 0, 0)),
            pl.BlockSpec((B, DIM), lambda g, s: (0, 0)),
        ],
        out_specs=[
            pl.BlockSpec((BK, 2 * N, DIM), lambda g, s: (g, 0, 0)),
            pl.BlockSpec((BK, DIM), lambda g, s: (g, 0)),
        ],
    )
    return pl.pallas_call(
        _bigcopy_body,
        grid_spec=grid_spec,
        out_shape=[
            jax.ShapeDtypeStruct((R, 2 * N, DIM), jnp.float32),
            jax.ShapeDtypeStruct((R, DIM), jnp.float32),
        ],
    )(ridx, qg3, q2d, grid_src, tclsB)


def _loss_body(val_ref, q2_ref, tr_ref, tf_ref, sr_ref, sf_ref, cg_ref,
               out_ref, acc_ref):
    g = pl.program_id(0)

    @pl.when(g == 0)
    def _():
        acc_ref[0] = 0.0

    vb = val_ref[g] != 0
    q2 = q2_ref[0]
    ret2 = jnp.where(vb, q2[:N], tr_ref[0])
    ret3 = jnp.where(vb, q2[N:], tf_ref[0])

    x = (ret2 - cg_ref[...]) / TAU
    x = x - jnp.max(x, axis=-1, keepdims=True)
    e = jnp.exp(x)
    t_region = e / jnp.sum(e, axis=-1, keepdims=True)

    tn = ret3 / jnp.maximum(
        jnp.sqrt(jnp.sum(ret3 * ret3, axis=-1, keepdims=True)), 1e-12)
    sfv = sf_ref[0]
    sn = sfv / jnp.maximum(
        jnp.sqrt(jnp.sum(sfv * sfv, axis=-1, keepdims=True)), 1e-12)
    sim = lax.dot_general(sn, tn, (((1,), (1,)), ((), ())),
                          preferred_element_type=jnp.float32)
    mx = jnp.max(sim, axis=1, keepdims=True)
    ii = lax.broadcasted_iota(jnp.int32, sim.shape, 1)
    ind = jnp.min(jnp.where(sim == mx, ii, N), axis=1, keepdims=True)
    oh = (ind == lax.broadcasted_iota(jnp.int32, (N, N), 1)).astype(jnp.float32)
    t_indexed = jnp.dot(oh, t_region, preferred_element_type=jnp.float32)

    s = sr_ref[0] / TAU_STU
    ls = s - jnp.max(s, axis=-1, keepdims=True)
    ls = ls - jnp.log(jnp.sum(jnp.exp(ls), axis=-1, keepdims=True))
    acc_ref[0] += 0.5 * jnp.mean(jnp.sum(-t_indexed * ls, axis=-1))

    @pl.when(g == pl.num_programs(0) - 1)
    def _():
        out_ref[...] = jnp.full((1, 1), acc_ref[0] / (2.0 * B), jnp.float32)


def _loss(valid, q2, t_region_out, t_fea, s_region_out, s_fea, center_grid):
    row = pl.BlockSpec((1, N, DIM), lambda g, s: (g, 0, 0))
    srow = pl.BlockSpec((1, N, DIM), lambda g, s: ((g + B) % (2 * B), 0, 0))
    grid_spec = pltpu.PrefetchScalarGridSpec(
        num_scalar_prefetch=1,
        grid=(2 * B,),
        in_specs=[
            pl.BlockSpec((1, 2 * N, DIM), lambda g, s: (g, 0, 0)),
            row, row, srow, srow,
            pl.BlockSpec((1, DIM), lambda g, s: (0, 0)),
        ],
        out_specs=pl.BlockSpec((1, 1), lambda g, s: (0, 0)),
        scratch_shapes=[pltpu.SMEM((1,), jnp.float32)],
    )
    out = pl.pallas_call(
        _loss_body,
        grid_spec=grid_spec,
        out_shape=jax.ShapeDtypeStruct((1, 1), jnp.float32),
    )(valid, q2, t_region_out, t_fea, s_region_out, s_fea, center_grid)
    return out[0, 0]


def kernel(t_cls_out, t_region_out, t_fea, s_cls_out, s_region_out, s_fea,
           concept, queue, queue_grid, queue_ptr, center, center_grid):
    key = jax.random.key(42)
    k1, k2, k3 = jax.random.split(key, 3)
    g2 = jax.random.gumbel(k1, (2 * B, C), jnp.float32)
    u = jax.random.uniform(k2, (2 * B,))
    gB = jax.random.gumbel(k3, (B, C), jnp.float32)

    r2, valid, ridx, new_ptr = _route(concept, queue_ptr, g2, u, gB)

    qg3 = queue_grid.reshape(R, 2 * N, DIM)
    q2d = queue.reshape(R, DIM)
    grid_src = jnp.concatenate([t_region_out[:B], t_fea[:B]], axis=1)

    q2rows = _sc_gather(queue_grid.reshape(R, ROW),
                        r2.reshape(NW_GATHER, (2 * B) // NW_GATHER))

    new_qg, new_q = _bigcopy(ridx.reshape(B), qg3, q2d, grid_src, t_cls_out[:B])

    total_loss = _loss(valid.reshape(2 * B), q2rows.reshape(2 * B, 2 * N, DIM),
                       t_region_out, t_fea, s_region_out, s_fea, center_grid)

    return (total_loss,
            new_q.reshape(C, K, DIM),
            new_qg.reshape(C, K, 2 * N, DIM),
            new_ptr)


# speedup vs baseline: 1.2328x; 1.2328x over previous
import functools

import jax
import jax.numpy as jnp
from jax import lax
from jax.experimental import pallas as pl
from jax.experimental.pallas import tpu as pltpu
from jax.experimental.pallas import tpu_sc as plsc

B, N, DIM, C, K = 32, 49, 128, 32, 128
TAU, TAU_STU = 0.04, 0.1
R = C * K
ROW = 2 * N * DIM
BK = 8
NW_GATHER = 8


def _argmax_first(x, size):
    m = jnp.max(x, axis=1, keepdims=True)
    ii = lax.broadcasted_iota(jnp.int32, x.shape, 1)
    return jnp.min(jnp.where(x == m, ii, size), axis=1, keepdims=True)


def _route_body(c_ref, ptr_ref, g2_ref, u_ref, gB_ref,
                r2_ref, val_ref, ridx_ref, nptr_ref):
    cpt = c_ref[...]
    ptrf = ptr_ref[...]

    cpt2 = jnp.concatenate([cpt, cpt], axis=0)
    mask2 = jnp.sum(cpt2, axis=1, keepdims=True) == 0
    logits2 = jnp.log(cpt2 + mask2.astype(jnp.float32) + 1e-12) + g2_ref[...]
    cs2 = _argmax_first(logits2, C)
    oh2 = (cs2 == lax.broadcasted_iota(jnp.int32, (2 * B, C), 1)).astype(jnp.float32)
    p2 = jnp.dot(oh2, ptrf, preferred_element_type=jnp.float32)
    size2 = jnp.clip(p2[:, 1:2], 0.0, float(K))
    pos2 = jnp.minimum(
        jnp.floor(u_ref[...] * jnp.maximum(size2, 1.0)).astype(jnp.int32), K - 1)
    valid = (size2 != 0.0) & jnp.logical_not(mask2)
    r2_ref[...] = cs2 * K + pos2
    val_ref[...] = valid.astype(jnp.int32)

    maskB = jnp.sum(cpt, axis=1, keepdims=True) == 0
    logitsB = jnp.log(cpt + maskB.astype(jnp.float32) + 1e-12) + gB_ref[...]
    csB = _argmax_first(logitsB, C)
    ohB = (csB == lax.broadcasted_iota(jnp.int32, (B, C), 1)).astype(jnp.float32)
    pB = jnp.dot(ohB, ptrf, preferred_element_type=jnp.float32)
    ptrB0 = pB[:, 0:1].astype(jnp.int32)
    ptrB1f = pB[:, 1:2]
    posB = lax.rem(ptrB0 + 1, K)
    do = jnp.logical_not(maskB)
    rB = csB * K + posB

    S = (lax.broadcasted_iota(jnp.int32, (B, B), 0)
         < lax.broadcasted_iota(jnp.int32, (B, B), 1)).astype(jnp.float32)
    ohR = (rB == lax.broadcasted_iota(jnp.int32, (B, R), 1)).astype(jnp.float32)
    later = jnp.dot(S, ohR, preferred_element_type=jnp.float32)
    eff = do & (jnp.sum(ohR * later, axis=1, keepdims=True) == 0)
    laterB = jnp.dot(S, ohB, preferred_element_type=jnp.float32)
    effp = do & (jnp.sum(ohB * laterB, axis=1, keepdims=True) == 0)
    ridx_ref[...] = jnp.where(eff, rB, -1)

    w = ohB * effp.astype(jnp.float32)
    dn = (((0,), (0,)), ((), ()))
    hit = lax.dot_general(w, jnp.ones((B, 1), jnp.float32), dn,
                          preferred_element_type=jnp.float32)
    n0 = lax.dot_general(w, posB.astype(jnp.float32), dn,
                         preferred_element_type=jnp.float32)
    v1 = jnp.clip(ptrB1f + 1.0, 0.0, float(K))
    n1 = lax.dot_general(w, v1, dn, preferred_element_type=jnp.float32)
    new0 = jnp.where(hit > 0, n0, ptrf[:, 0:1])
    new1 = jnp.where(hit > 0, n1, ptrf[:, 1:2])
    nptr_ref[...] = jnp.concatenate([new0, new1], axis=1).astype(jnp.int32)


def _route(concept, queue_ptr, g2, u, gB):
    return pl.pallas_call(
        _route_body,
        out_shape=[
            jax.ShapeDtypeStruct((2 * B, 1), jnp.int32),
            jax.ShapeDtypeStruct((2 * B, 1), jnp.int32),
            jax.ShapeDtypeStruct((B, 1), jnp.int32),
            jax.ShapeDtypeStruct((C, 2), jnp.int32),
        ],
    )(concept, queue_ptr.astype(jnp.float32), g2, u[:, None], gB)


def _sc_gather(qg3, idx2):
    info = plsc.get_sparse_core_info()
    rows_per_w = (2 * B) // NW_GATHER

    @functools.partial(
        pl.kernel,
        mesh=plsc.VectorSubcoreMesh(core_axis_name="c", subcore_axis_name="s"),
        out_type=jax.ShapeDtypeStruct((2 * B, 2 * N, DIM), jnp.float32),
        scratch_types=[
            pltpu.VMEM((rows_per_w,), jnp.int32),
            pltpu.VMEM((rows_per_w, 2 * N, DIM), jnp.float32),
            pltpu.SemaphoreType.DMA,
        ],
    )
    def gather(qg_ref, idx_ref, out_ref, idx_v, rows_v, sem):
        wid = lax.axis_index("s") * info.num_cores + lax.axis_index("c")

        @pl.when(wid < NW_GATHER)
        def _():
            pltpu.sync_copy(idx_ref.at[wid], idx_v)
            pltpu.async_copy(qg_ref.at[idx_v], rows_v, sem).wait()
            pltpu.sync_copy(rows_v, out_ref.at[pl.ds(wid * rows_per_w, rows_per_w)])

    return gather(qg3, idx2)


def _bigcopy_body(ridx_ref, qg_ref, q_ref, gs_ref, tcls_ref, og_ref, oq_ref):
    og_ref[...] = qg_ref[...]
    oq_ref[...] = q_ref[...]
    base = pl.program_id(0) * BK
    for i in range(B):
        r = ridx_ref[i]

        @pl.when((r >= base) & (r < base + BK))
        def _(i=i, r=r):
            k = r - base
            og_ref[pl.ds(k, 1)] = gs_ref[i][None]
            oq_ref[pl.ds(k, 1)] = tcls_ref[i][None]


def _bigcopy(ridx, qg3, q2d, grid_src, tclsB):
    grid_spec = pltpu.PrefetchScalarGridSpec(
        num_scalar_prefetch=1,
        grid=(R // BK,),
        in_specs=[
            pl.BlockSpec((BK, 2 * N, DIM), lambda g, s: (g, 0, 0)),
            pl.BlockSpec((BK, DIM), lambda g, s: (g, 0)),
            pl.BlockSpec((B, 2 * N, DIM), lambda g, s: (0, 0, 0)),
            pl.BlockSpec((B, DIM), lambda g, s: (0, 0)),
        ],
        out_specs=[
            pl.BlockSpec((BK, 2 * N, DIM), lambda g, s: (g, 0, 0)),
            pl.BlockSpec((BK, DIM), lambda g, s: (g, 0)),
        ],
    )
    return pl.pallas_call(
        _bigcopy_body,
        grid_spec=grid_spec,
        out_shape=[
            jax.ShapeDtypeStruct((R, 2 * N, DIM), jnp.float32),
            jax.ShapeDtypeStruct((R, DIM), jnp.float32),
        ],
    )(ridx, qg3, q2d, grid_src, tclsB)


def _loss_body(val_ref, q2_ref, tr_ref, tf_ref, sr_ref, sf_ref, cg_ref,
               out_ref, acc_ref):
    g = pl.program_id(0)

    @pl.when(g == 0)
    def _():
        acc_ref[0] = 0.0

    vb = val_ref[g] != 0
    q2 = q2_ref[0]
    ret2 = jnp.where(vb, q2[:N], tr_ref[0])
    ret3 = jnp.where(vb, q2[N:], tf_ref[0])

    x = (ret2 - cg_ref[...]) / TAU
    x = x - jnp.max(x, axis=-1, keepdims=True)
    e = jnp.exp(x)
    t_region = e / jnp.sum(e, axis=-1, keepdims=True)

    tn = ret3 / jnp.maximum(
        jnp.sqrt(jnp.sum(ret3 * ret3, axis=-1, keepdims=True)), 1e-12)
    sfv = sf_ref[0]
    sn = sfv / jnp.maximum(
        jnp.sqrt(jnp.sum(sfv * sfv, axis=-1, keepdims=True)), 1e-12)
    sim = lax.dot_general(sn, tn, (((1,), (1,)), ((), ())),
                          preferred_element_type=jnp.float32)
    mx = jnp.max(sim, axis=1, keepdims=True)
    ii = lax.broadcasted_iota(jnp.int32, sim.shape, 1)
    ind = jnp.min(jnp.where(sim == mx, ii, N), axis=1, keepdims=True)
    oh = (ind == lax.broadcasted_iota(jnp.int32, (N, N), 1)).astype(jnp.float32)
    t_indexed = jnp.dot(oh, t_region, preferred_element_type=jnp.float32)

    s = sr_ref[0] / TAU_STU
    ls = s - jnp.max(s, axis=-1, keepdims=True)
    ls = ls - jnp.log(jnp.sum(jnp.exp(ls), axis=-1, keepdims=True))
    acc_ref[0] += 0.5 * jnp.mean(jnp.sum(-t_indexed * ls, axis=-1))

    @pl.when(g == pl.num_programs(0) - 1)
    def _():
        out_ref[...] = jnp.full((1, 1), acc_ref[0] / (2.0 * B), jnp.float32)


def _loss(valid, q2, t_region_out, t_fea, s_region_out, s_fea, center_grid):
    row = pl.BlockSpec((1, N, DIM), lambda g, s: (g, 0, 0))
    srow = pl.BlockSpec((1, N, DIM), lambda g, s: ((g + B) % (2 * B), 0, 0))
    grid_spec = pltpu.PrefetchScalarGridSpec(
        num_scalar_prefetch=1,
        grid=(2 * B,),
        in_specs=[
            pl.BlockSpec((1, 2 * N, DIM), lambda g, s: (g, 0, 0)),
            row, row, srow, srow,
            pl.BlockSpec((1, DIM), lambda g, s: (0, 0)),
        ],
        out_specs=pl.BlockSpec((1, 1), lambda g, s: (0, 0)),
        scratch_shapes=[pltpu.SMEM((1,), jnp.float32)],
    )
    out = pl.pallas_call(
        _loss_body,
        grid_spec=grid_spec,
        out_shape=jax.ShapeDtypeStruct((1, 1), jnp.float32),
    )(valid, q2, t_region_out, t_fea, s_region_out, s_fea, center_grid)
    return out[0, 0]


def kernel(t_cls_out, t_region_out, t_fea, s_cls_out, s_region_out, s_fea,
           concept, queue, queue_grid, queue_ptr, center, center_grid):
    key = jax.random.key(42)
    k1, k2, k3 = jax.random.split(key, 3)
    g2 = jax.random.gumbel(k1, (2 * B, C), jnp.float32)
    u = jax.random.uniform(k2, (2 * B,))
    gB = jax.random.gumbel(k3, (B, C), jnp.float32)

    r2, valid, ridx, new_ptr = _route(concept, queue_ptr, g2, u, gB)

    qg3 = queue_grid.reshape(R, 2 * N, DIM)
    q2d = queue.reshape(R, DIM)
    grid_src = jnp.concatenate([t_region_out[:B], t_fea[:B]], axis=1)

    q2rows = _sc_gather(qg3, r2.reshape(NW_GATHER, (2 * B) // NW_GATHER))

    new_qg, new_q = _bigcopy(ridx.reshape(B), qg3, q2d, grid_src, t_cls_out[:B])

    total_loss = _loss(valid.reshape(2 * B), q2rows,
                       t_region_out, t_fea, s_region_out, s_fea, center_grid)

    return (total_loss,
            new_q.reshape(C, K, DIM),
            new_qg.reshape(C, K, 2 * N, DIM),
            new_ptr)


# speedup vs baseline: 1.2435x; 1.0087x over previous
import functools

import jax
import jax.numpy as jnp
from jax import lax
from jax.experimental import pallas as pl
from jax.experimental.pallas import tpu as pltpu
from jax.experimental.pallas import tpu_sc as plsc

B, N, DIM, C, K = 32, 49, 128, 32, 128
TAU, TAU_STU = 0.04, 0.1
R = C * K
ROW = 2 * N * DIM
BK = 8
NW_GATHER = 8


def _argmax_first(x, size):
    m = jnp.max(x, axis=1, keepdims=True)
    ii = lax.broadcasted_iota(jnp.int32, x.shape, 1)
    return jnp.min(jnp.where(x == m, ii, size), axis=1, keepdims=True)


def _route_body(c_ref, ptr_ref, g2_ref, u_ref, gB_ref,
                r2_ref, val_ref, ridx_ref, nptr_ref):
    cpt = c_ref[...]
    ptrf = ptr_ref[...]

    cpt2 = jnp.concatenate([cpt, cpt], axis=0)
    mask2 = jnp.sum(cpt2, axis=1, keepdims=True) == 0
    logits2 = jnp.log(cpt2 + mask2.astype(jnp.float32) + 1e-12) + g2_ref[...]
    cs2 = _argmax_first(logits2, C)
    oh2 = (cs2 == lax.broadcasted_iota(jnp.int32, (2 * B, C), 1)).astype(jnp.float32)
    p2 = jnp.dot(oh2, ptrf, preferred_element_type=jnp.float32)
    size2 = jnp.clip(p2[:, 1:2], 0.0, float(K))
    pos2 = jnp.minimum(
        jnp.floor(u_ref[...] * jnp.maximum(size2, 1.0)).astype(jnp.int32), K - 1)
    valid = (size2 != 0.0) & jnp.logical_not(mask2)
    r2_ref[...] = cs2 * K + pos2
    val_ref[...] = valid.astype(jnp.int32)

    maskB = jnp.sum(cpt, axis=1, keepdims=True) == 0
    logitsB = jnp.log(cpt + maskB.astype(jnp.float32) + 1e-12) + gB_ref[...]
    csB = _argmax_first(logitsB, C)
    ohB = (csB == lax.broadcasted_iota(jnp.int32, (B, C), 1)).astype(jnp.float32)
    pB = jnp.dot(ohB, ptrf, preferred_element_type=jnp.float32)
    ptrB0 = pB[:, 0:1].astype(jnp.int32)
    ptrB1f = pB[:, 1:2]
    posB = lax.rem(ptrB0 + 1, K)
    do = jnp.logical_not(maskB)
    rB = csB * K + posB

    S = (lax.broadcasted_iota(jnp.int32, (B, B), 0)
         < lax.broadcasted_iota(jnp.int32, (B, B), 1)).astype(jnp.float32)
    ohR = (rB == lax.broadcasted_iota(jnp.int32, (B, R), 1)).astype(jnp.float32)
    later = jnp.dot(S, ohR, preferred_element_type=jnp.float32)
    eff = do & (jnp.sum(ohR * later, axis=1, keepdims=True) == 0)
    laterB = jnp.dot(S, ohB, preferred_element_type=jnp.float32)
    effp = do & (jnp.sum(ohB * laterB, axis=1, keepdims=True) == 0)
    ridx_ref[...] = jnp.where(eff, rB, -1)

    w = ohB * effp.astype(jnp.float32)
    dn = (((0,), (0,)), ((), ()))
    hit = lax.dot_general(w, jnp.ones((B, 1), jnp.float32), dn,
                          preferred_element_type=jnp.float32)
    n0 = lax.dot_general(w, posB.astype(jnp.float32), dn,
                         preferred_element_type=jnp.float32)
    v1 = jnp.clip(ptrB1f + 1.0, 0.0, float(K))
    n1 = lax.dot_general(w, v1, dn, preferred_element_type=jnp.float32)
    new0 = jnp.where(hit > 0, n0, ptrf[:, 0:1])
    new1 = jnp.where(hit > 0, n1, ptrf[:, 1:2])
    nptr_ref[...] = jnp.concatenate([new0, new1], axis=1).astype(jnp.int32)


def _route(concept, queue_ptr, g2, u, gB):
    return pl.pallas_call(
        _route_body,
        out_shape=[
            jax.ShapeDtypeStruct((2 * B, 1), jnp.int32),
            jax.ShapeDtypeStruct((2 * B, 1), jnp.int32),
            jax.ShapeDtypeStruct((B, 1), jnp.int32),
            jax.ShapeDtypeStruct((C, 2), jnp.int32),
        ],
    )(concept, queue_ptr.astype(jnp.float32), g2, u[:, None], gB)


def _sc_gather(qg3, idx2):
    info = plsc.get_sparse_core_info()
    rows_per_w = (2 * B) // NW_GATHER

    @functools.partial(
        pl.kernel,
        mesh=plsc.VectorSubcoreMesh(core_axis_name="c", subcore_axis_name="s"),
        out_type=jax.ShapeDtypeStruct((2 * B, 2 * N, DIM), jnp.float32),
        scratch_types=[
            pltpu.VMEM((rows_per_w,), jnp.int32),
            pltpu.VMEM((rows_per_w, 2 * N, DIM), jnp.float32),
            pltpu.SemaphoreType.DMA,
        ],
    )
    def gather(qg_ref, idx_ref, out_ref, idx_v, rows_v, sem):
        wid = lax.axis_index("s") * info.num_cores + lax.axis_index("c")

        @pl.when(wid < NW_GATHER)
        def _():
            pltpu.sync_copy(idx_ref.at[wid], idx_v)
            pltpu.async_copy(qg_ref.at[idx_v], rows_v, sem).wait()
            pltpu.sync_copy(rows_v, out_ref.at[pl.ds(wid * rows_per_w, rows_per_w)])

    return gather(qg3, idx2)


def _bigcopy_body(ridx_ref, qg_ref, q_ref, gs_ref, tcls_ref, og_ref, oq_ref):
    og_ref[...] = qg_ref[...]
    oq_ref[...] = q_ref[...]
    base = pl.program_id(0) * BK
    for i in range(B):
        r = ridx_ref[i]

        @pl.when((r >= base) & (r < base + BK))
        def _(i=i, r=r):
            k = r - base
            og_ref[pl.ds(k, 1)] = gs_ref[i][None]
            oq_ref[pl.ds(k, 1)] = tcls_ref[i][None]


def _bigcopy(ridx, qg3, q2d, grid_src, tclsB):
    grid_spec = pltpu.PrefetchScalarGridSpec(
        num_scalar_prefetch=1,
        grid=(R // BK,),
        in_specs=[
            pl.BlockSpec((BK, 2 * N, DIM), lambda g, s: (g, 0, 0)),
            pl.BlockSpec((BK, DIM), lambda g, s: (g, 0)),
            pl.BlockSpec((B, 2 * N, DIM), lambda g, s: (0, 0, 0)),
            pl.BlockSpec((B, DIM), lambda g, s: (0, 0)),
        ],
        out_specs=[
            pl.BlockSpec((BK, 2 * N, DIM), lambda g, s: (g, 0, 0)),
            pl.BlockSpec((BK, DIM), lambda g, s: (g, 0)),
        ],
    )
    return pl.pallas_call(
        _bigcopy_body,
        grid_spec=grid_spec,
        out_shape=[
            jax.ShapeDtypeStruct((R, 2 * N, DIM), jnp.float32),
            jax.ShapeDtypeStruct((R, DIM), jnp.float32),
        ],
    )(ridx, qg3, q2d, grid_src, tclsB)


def _loss_body(val_ref, r2_ref, q2_ref, tr_ref, tf_ref, sr_ref, sf_ref, cg_ref,
               out_ref, acc_ref):
    g = pl.program_id(0)

    @pl.when(g == 0)
    def _():
        acc_ref[0] = 0.0

    vb = val_ref[g] != 0
    q2 = q2_ref[0]
    ret2 = jnp.where(vb, q2[:N], tr_ref[0])
    ret3 = jnp.where(vb, q2[N:], tf_ref[0])

    x = (ret2 - cg_ref[...]) / TAU
    x = x - jnp.max(x, axis=-1, keepdims=True)
    e = jnp.exp(x)
    t_region = e / jnp.sum(e, axis=-1, keepdims=True)

    tn = ret3 / jnp.maximum(
        jnp.sqrt(jnp.sum(ret3 * ret3, axis=-1, keepdims=True)), 1e-12)
    sfv = sf_ref[0]
    sn = sfv / jnp.maximum(
        jnp.sqrt(jnp.sum(sfv * sfv, axis=-1, keepdims=True)), 1e-12)
    sim = lax.dot_general(sn, tn, (((1,), (1,)), ((), ())),
                          preferred_element_type=jnp.float32)
    mx = jnp.max(sim, axis=1, keepdims=True)
    ii = lax.broadcasted_iota(jnp.int32, sim.shape, 1)
    ind = jnp.min(jnp.where(sim == mx, ii, N), axis=1, keepdims=True)
    oh = (ind == lax.broadcasted_iota(jnp.int32, (N, N), 1)).astype(jnp.float32)
    t_indexed = jnp.dot(oh, t_region, preferred_element_type=jnp.float32)

    s = sr_ref[0] / TAU_STU
    ls = s - jnp.max(s, axis=-1, keepdims=True)
    ls = ls - jnp.log(jnp.sum(jnp.exp(ls), axis=-1, keepdims=True))
    acc_ref[0] += 0.5 * jnp.mean(jnp.sum(-t_indexed * ls, axis=-1))

    @pl.when(g == pl.num_programs(0) - 1)
    def _():
        out_ref[...] = jnp.full((1, 1), acc_ref[0] / (2.0 * B), jnp.float32)


def _loss(valid, r2, qg3, t_region_out, t_fea, s_region_out, s_fea,
          center_grid):
    row = pl.BlockSpec((1, N, DIM), lambda g, v, r: (g, 0, 0))
    srow = pl.BlockSpec((1, N, DIM), lambda g, v, r: ((g + B) % (2 * B), 0, 0))
    grid_spec = pltpu.PrefetchScalarGridSpec(
        num_scalar_prefetch=2,
        grid=(2 * B,),
        in_specs=[
            pl.BlockSpec((1, 2 * N, DIM), lambda g, v, r: (r[g], 0, 0)),
            row, row, srow, srow,
            pl.BlockSpec((1, DIM), lambda g, v, r: (0, 0)),
        ],
        out_specs=pl.BlockSpec((1, 1), lambda g, v, r: (0, 0)),
        scratch_shapes=[pltpu.SMEM((1,), jnp.float32)],
    )
    out = pl.pallas_call(
        _loss_body,
        grid_spec=grid_spec,
        out_shape=jax.ShapeDtypeStruct((1, 1), jnp.float32),
    )(valid, r2, qg3, t_region_out, t_fea, s_region_out, s_fea, center_grid)
    return out[0, 0]


def kernel(t_cls_out, t_region_out, t_fea, s_cls_out, s_region_out, s_fea,
           concept, queue, queue_grid, queue_ptr, center, center_grid):
    key = jax.random.key(42)
    k1, k2, k3 = jax.random.split(key, 3)
    g2 = jax.random.gumbel(k1, (2 * B, C), jnp.float32)
    u = jax.random.uniform(k2, (2 * B,))
    gB = jax.random.gumbel(k3, (B, C), jnp.float32)

    r2, valid, ridx, new_ptr = _route(concept, queue_ptr, g2, u, gB)

    qg3 = queue_grid.reshape(R, 2 * N, DIM)
    q2d = queue.reshape(R, DIM)
    grid_src = jnp.concatenate([t_region_out[:B], t_fea[:B]], axis=1)

    new_qg, new_q = _bigcopy(ridx.reshape(B), qg3, q2d, grid_src, t_cls_out[:B])

    total_loss = _loss(valid.reshape(2 * B), r2.reshape(2 * B), qg3,
                       t_region_out, t_fea, s_region_out, s_fea, center_grid)

    return (total_loss,
            new_q.reshape(C, K, DIM),
            new_qg.reshape(C, K, 2 * N, DIM),
            new_ptr)


# speedup vs baseline: 3.8757x; 3.1166x over previous
import functools

import jax
import jax.numpy as jnp
from jax import lax
from jax.experimental import pallas as pl
from jax.experimental.pallas import tpu as pltpu
from jax.experimental.pallas import tpu_sc as plsc

B, N, DIM, C, K = 32, 49, 128, 32, 128
TAU, TAU_STU = 0.04, 0.1
R = C * K


def _argmax_first(x, size):
    m = jnp.max(x, axis=1, keepdims=True)
    ii = lax.broadcasted_iota(jnp.int32, x.shape, 1)
    return jnp.min(jnp.where(x == m, ii, size), axis=1, keepdims=True)


def _route_body(c_ref, ptr_ref, g2_ref, u_ref, gB_ref, q2d_ref, tcls_ref,
                r2_ref, val_ref, ridx_ref, nptr_ref, scv_ref, rb_ref):
    cpt = c_ref[...]
    ptrf = ptr_ref[...]

    cpt2 = jnp.concatenate([cpt, cpt], axis=0)
    mask2 = jnp.sum(cpt2, axis=1, keepdims=True) == 0
    logits2 = jnp.log(cpt2 + mask2.astype(jnp.float32) + 1e-12) + g2_ref[...]
    cs2 = _argmax_first(logits2, C)
    oh2 = (cs2 == lax.broadcasted_iota(jnp.int32, (2 * B, C), 1)).astype(jnp.float32)
    p2 = jnp.dot(oh2, ptrf, preferred_element_type=jnp.float32)
    size2 = jnp.clip(p2[:, 1:2], 0.0, float(K))
    pos2 = jnp.minimum(
        jnp.floor(u_ref[...] * jnp.maximum(size2, 1.0)).astype(jnp.int32), K - 1)
    valid = (size2 != 0.0) & jnp.logical_not(mask2)
    r2_ref[...] = cs2 * K + pos2
    val_ref[...] = valid.astype(jnp.int32)

    maskB = jnp.sum(cpt, axis=1, keepdims=True) == 0
    logitsB = jnp.log(cpt + maskB.astype(jnp.float32) + 1e-12) + gB_ref[...]
    csB = _argmax_first(logitsB, C)
    ohB = (csB == lax.broadcasted_iota(jnp.int32, (B, C), 1)).astype(jnp.float32)
    pB = jnp.dot(ohB, ptrf, preferred_element_type=jnp.float32)
    ptrB0 = pB[:, 0:1].astype(jnp.int32)
    ptrB1f = pB[:, 1:2]
    posB = lax.rem(ptrB0 + 1, K)
    do = jnp.logical_not(maskB)
    rB = csB * K + posB

    S = (lax.broadcasted_iota(jnp.int32, (B, B), 0)
         < lax.broadcasted_iota(jnp.int32, (B, B), 1)).astype(jnp.float32)
    ohR = (rB == lax.broadcasted_iota(jnp.int32, (B, R), 1)).astype(jnp.float32)
    later = jnp.dot(S, ohR, preferred_element_type=jnp.float32)
    eff = do & (jnp.sum(ohR * later, axis=1, keepdims=True) == 0)
    laterB = jnp.dot(S, ohB, preferred_element_type=jnp.float32)
    effp = do & (jnp.sum(ohB * laterB, axis=1, keepdims=True) == 0)
    ridx_ref[...] = jnp.where(eff, rB, -1)
    rb_ref[...] = rB

    dn = (((1,), (1,)), ((), ()))

    M = lax.dot_general(ohR, ohR, dn, preferred_element_type=jnp.float32)
    E = lax.dot_general(jnp.ones((B, 1), jnp.float32), eff.astype(jnp.float32),
                        dn, preferred_element_type=jnp.float32)
    Wfin = M * E
    valsW = jnp.dot(Wfin, tcls_ref[...], preferred_element_type=jnp.float32)
    rowhas = jnp.sum(Wfin, axis=1, keepdims=True)
    orig = jnp.dot(ohR, q2d_ref[...], preferred_element_type=jnp.float32)
    scv_ref[...] = valsW + (1.0 - rowhas) * orig

    dn0 = (((0,), (0,)), ((), ()))
    w = ohB * effp.astype(jnp.float32)
    hit = lax.dot_general(w, jnp.ones((B, 1), jnp.float32), dn0,
                          preferred_element_type=jnp.float32)
    n0 = lax.dot_general(w, posB.astype(jnp.float32), dn0,
                         preferred_element_type=jnp.float32)
    v1 = jnp.clip(ptrB1f + 1.0, 0.0, float(K))
    n1 = lax.dot_general(w, v1, dn0, preferred_element_type=jnp.float32)
    new0 = jnp.where(hit > 0, n0, ptrf[:, 0:1])
    new1 = jnp.where(hit > 0, n1, ptrf[:, 1:2])
    nptr_ref[...] = jnp.concatenate([new0, new1], axis=1).astype(jnp.int32)


def _route(concept, queue_ptr, g2, u, gB, q2d, tclsB):
    return pl.pallas_call(
        _route_body,
        out_shape=[
            jax.ShapeDtypeStruct((2 * B, 1), jnp.int32),
            jax.ShapeDtypeStruct((2 * B, 1), jnp.int32),
            jax.ShapeDtypeStruct((B, 1), jnp.int32),
            jax.ShapeDtypeStruct((C, 2), jnp.int32),
            jax.ShapeDtypeStruct((B, DIM), jnp.float32),
            jax.ShapeDtypeStruct((B, 1), jnp.int32),
        ],
    )(concept, queue_ptr.astype(jnp.float32), g2, u[:, None], gB, q2d, tclsB)


def _sc_queue(q2d, scvals, rb):
    info = plsc.get_sparse_core_info()
    nsub = info.num_subcores
    rows_per = R // nsub

    @functools.partial(
        pl.kernel,
        mesh=plsc.VectorSubcoreMesh(core_axis_name="c", subcore_axis_name="s"),
        out_type=jax.ShapeDtypeStruct((R, DIM), jnp.float32),
        scratch_types=[
            pltpu.VMEM((rows_per, DIM), jnp.float32),
            pltpu.VMEM((B,), jnp.int32),
            pltpu.VMEM((B, DIM), jnp.float32),
            pltpu.SemaphoreType.DMA,
        ],
    )
    def qk(q_ref, vals_ref, idx_ref, out_ref, rows_v, idx_v, vals_v, sem):
        cid = lax.axis_index("c")
        sid = lax.axis_index("s")

        @pl.when(cid == 0)
        def _():
            base = sid * rows_per
            pltpu.sync_copy(q_ref.at[pl.ds(base, rows_per)], rows_v)
            pltpu.sync_copy(rows_v, out_ref.at[pl.ds(base, rows_per)])
            plsc.subcore_barrier()

            @pl.when(sid == 0)
            def _():
                pltpu.sync_copy(idx_ref, idx_v)
                pltpu.sync_copy(vals_ref, vals_v)
                pltpu.async_copy(vals_v, out_ref.at[idx_v], sem).wait()

    return qk(q2d, scvals, rb)


def _bigcopy_body(ridx_ref, r2_ref, qg_ref, gs_ref, og_ref, q2_ref):
    og_ref[...] = qg_ref[...]
    base = pl.program_id(0) * K
    for i in range(B):
        r = ridx_ref[i]

        @pl.when((r >= base) & (r < base + K))
        def _(i=i, r=r):
            og_ref[0, :, pl.ds(r - base, 1), :] = gs_ref[:, i, :][:, None, :]

    for j in range(2 * B):
        rj = r2_ref[j]

        @pl.when((rj >= base) & (rj < base + K))
        def _(j=j, rj=rj):
            q2_ref[j] = qg_ref[0, :, pl.ds(rj - base, 1), :].reshape(2 * N, DIM)


def _bigcopy(ridx, r2, qgT, gsT):
    grid_spec = pltpu.PrefetchScalarGridSpec(
        num_scalar_prefetch=2,
        grid=(C,),
        in_specs=[
            pl.BlockSpec((1, 2 * N, K, DIM), lambda g, s1, s2: (g, 0, 0, 0)),
            pl.BlockSpec((2 * N, B, DIM), lambda g, s1, s2: (0, 0, 0)),
        ],
        out_specs=[
            pl.BlockSpec((1, 2 * N, K, DIM), lambda g, s1, s2: (g, 0, 0, 0)),
            pl.BlockSpec((2 * B, 2 * N, DIM), lambda g, s1, s2: (0, 0, 0)),
        ],
    )
    return pl.pallas_call(
        _bigcopy_body,
        grid_spec=grid_spec,
        out_shape=[
            jax.ShapeDtypeStruct((C, 2 * N, K, DIM), jnp.float32),
            jax.ShapeDtypeStruct((2 * B, 2 * N, DIM), jnp.float32),
        ],
    )(ridx, r2, qgT, gsT)


def _loss_body(val_ref, q2_ref, tr_ref, tf_ref, sr_ref, sf_ref, cg_ref,
               out_ref, acc_ref):
    g = pl.program_id(0)

    @pl.when(g == 0)
    def _():
        acc_ref[0] = 0.0

    vb = val_ref[g] != 0
    q2 = q2_ref[0]
    ret2 = jnp.where(vb, q2[:N], tr_ref[0])
    ret3 = jnp.where(vb, q2[N:], tf_ref[0])

    x = (ret2 - cg_ref[...]) / TAU
    x = x - jnp.max(x, axis=-1, keepdims=True)
    e = jnp.exp(x)
    t_region = e / jnp.sum(e, axis=-1, keepdims=True)

    tn = ret3 / jnp.maximum(
        jnp.sqrt(jnp.sum(ret3 * ret3, axis=-1, keepdims=True)), 1e-12)
    sfv = sf_ref[0]
    sn = sfv / jnp.maximum(
        jnp.sqrt(jnp.sum(sfv * sfv, axis=-1, keepdims=True)), 1e-12)
    sim = lax.dot_general(sn, tn, (((1,), (1,)), ((), ())),
                          preferred_element_type=jnp.float32)
    mx = jnp.max(sim, axis=1, keepdims=True)
    ii = lax.broadcasted_iota(jnp.int32, sim.shape, 1)
    ind = jnp.min(jnp.where(sim == mx, ii, N), axis=1, keepdims=True)
    oh = (ind == lax.broadcasted_iota(jnp.int32, (N, N), 1)).astype(jnp.float32)
    t_indexed = jnp.dot(oh, t_region, preferred_element_type=jnp.float32)

    s = sr_ref[0] / TAU_STU
    ls = s - jnp.max(s, axis=-1, keepdims=True)
    ls = ls - jnp.log(jnp.sum(jnp.exp(ls), axis=-1, keepdims=True))
    acc_ref[0] += 0.5 * jnp.mean(jnp.sum(-t_indexed * ls, axis=-1))

    @pl.when(g == pl.num_programs(0) - 1)
    def _():
        out_ref[...] = jnp.full((1, 1), acc_ref[0] / (2.0 * B), jnp.float32)


def _loss(valid, q2, t_region_out, t_fea, s_region_out, s_fea, center_grid):
    row = pl.BlockSpec((1, N, DIM), lambda g, v: (g, 0, 0))
    srow = pl.BlockSpec((1, N, DIM), lambda g, v: ((g + B) % (2 * B), 0, 0))
    grid_spec = pltpu.PrefetchScalarGridSpec(
        num_scalar_prefetch=1,
        grid=(2 * B,),
        in_specs=[
            pl.BlockSpec((1, 2 * N, DIM), lambda g, v: (g, 0, 0)),
            row, row, srow, srow,
            pl.BlockSpec((1, DIM), lambda g, v: (0, 0)),
        ],
        out_specs=pl.BlockSpec((1, 1), lambda g, v: (0, 0)),
        scratch_shapes=[pltpu.SMEM((1,), jnp.float32)],
    )
    out = pl.pallas_call(
        _loss_body,
        grid_spec=grid_spec,
        out_shape=jax.ShapeDtypeStruct((1, 1), jnp.float32),
    )(valid, q2, t_region_out, t_fea, s_region_out, s_fea, center_grid)
    return out[0, 0]


def kernel(t_cls_out, t_region_out, t_fea, s_cls_out, s_region_out, s_fea,
           concept, queue, queue_grid, queue_ptr, center, center_grid):
    key = jax.random.key(42)
    k1, k2, k3 = jax.random.split(key, 3)
    g2 = jax.random.gumbel(k1, (2 * B, C), jnp.float32)
    u = jax.random.uniform(k2, (2 * B,))
    gB = jax.random.gumbel(k3, (B, C), jnp.float32)

    q2d = queue.reshape(R, DIM)
    r2, valid, ridx, new_ptr, scvals, rb = _route(
        concept, queue_ptr, g2, u, gB, q2d, t_cls_out[:B])

    new_q = _sc_queue(q2d, scvals, rb.reshape(B))

    qgT = queue_grid.transpose(0, 2, 1, 3)
    gsT = jnp.concatenate([t_region_out[:B].transpose(1, 0, 2),
                           t_fea[:B].transpose(1, 0, 2)], axis=0)

    new_qgT, q2rows = _bigcopy(ridx.reshape(B), r2.reshape(2 * B), qgT, gsT)

    total_loss = _loss(valid.reshape(2 * B), q2rows,
                       t_region_out, t_fea, s_region_out, s_fea, center_grid)

    return (total_loss,
            new_q.reshape(C, K, DIM),
            new_qgT.transpose(0, 2, 1, 3),
            new_ptr)


# speedup vs baseline: 4.5948x; 1.1856x over previous
import functools

import jax
import jax.numpy as jnp
from jax import lax
from jax.experimental import pallas as pl
from jax.experimental.pallas import tpu as pltpu
from jax.experimental.pallas import tpu_sc as plsc

B, N, DIM, C, K = 32, 49, 128, 32, 128
TAU, TAU_STU = 0.04, 0.1
R = C * K


def _argmax_first(x, size):
    m = jnp.max(x, axis=1, keepdims=True)
    ii = lax.broadcasted_iota(jnp.int32, x.shape, 1)
    return jnp.min(jnp.where(x == m, ii, size), axis=1, keepdims=True)


def _route_body(c_ref, ptr_ref, g2_ref, u_ref, gB_ref, q2d_ref, tcls_ref,
                r2_ref, val_ref, ridx_ref, nptr_ref, scv_ref, rb_ref):
    cpt = c_ref[...]
    ptrf = ptr_ref[...]

    cpt2 = jnp.concatenate([cpt, cpt], axis=0)
    mask2 = jnp.sum(cpt2, axis=1, keepdims=True) == 0
    logits2 = jnp.log(cpt2 + mask2.astype(jnp.float32) + 1e-12) + g2_ref[...]
    cs2 = _argmax_first(logits2, C)
    oh2 = (cs2 == lax.broadcasted_iota(jnp.int32, (2 * B, C), 1)).astype(jnp.float32)
    p2 = jnp.dot(oh2, ptrf, preferred_element_type=jnp.float32)
    size2 = jnp.clip(p2[:, 1:2], 0.0, float(K))
    pos2 = jnp.minimum(
        jnp.floor(u_ref[...] * jnp.maximum(size2, 1.0)).astype(jnp.int32), K - 1)
    valid = (size2 != 0.0) & jnp.logical_not(mask2)
    r2_ref[...] = cs2 * K + pos2
    val_ref[...] = valid.astype(jnp.int32)

    maskB = jnp.sum(cpt, axis=1, keepdims=True) == 0
    logitsB = jnp.log(cpt + maskB.astype(jnp.float32) + 1e-12) + gB_ref[...]
    csB = _argmax_first(logitsB, C)
    ohB = (csB == lax.broadcasted_iota(jnp.int32, (B, C), 1)).astype(jnp.float32)
    pB = jnp.dot(ohB, ptrf, preferred_element_type=jnp.float32)
    ptrB0 = pB[:, 0:1].astype(jnp.int32)
    ptrB1f = pB[:, 1:2]
    posB = lax.rem(ptrB0 + 1, K)
    do = jnp.logical_not(maskB)
    rB = csB * K + posB

    S = (lax.broadcasted_iota(jnp.int32, (B, B), 0)
         < lax.broadcasted_iota(jnp.int32, (B, B), 1)).astype(jnp.float32)
    ohR = (rB == lax.broadcasted_iota(jnp.int32, (B, R), 1)).astype(jnp.float32)
    later = jnp.dot(S, ohR, preferred_element_type=jnp.float32)
    eff = do & (jnp.sum(ohR * later, axis=1, keepdims=True) == 0)
    laterB = jnp.dot(S, ohB, preferred_element_type=jnp.float32)
    effp = do & (jnp.sum(ohB * laterB, axis=1, keepdims=True) == 0)
    ridx_ref[...] = jnp.where(eff, rB, -1)
    rb_ref[...] = rB

    dn = (((1,), (1,)), ((), ()))

    M = lax.dot_general(ohR, ohR, dn, preferred_element_type=jnp.float32)
    E = lax.dot_general(jnp.ones((B, 1), jnp.float32), eff.astype(jnp.float32),
                        dn, preferred_element_type=jnp.float32)
    Wfin = M * E
    valsW = jnp.dot(Wfin, tcls_ref[...], preferred_element_type=jnp.float32)
    rowhas = jnp.sum(Wfin, axis=1, keepdims=True)
    orig = jnp.dot(ohR, q2d_ref[...], preferred_element_type=jnp.float32)
    scv_ref[...] = valsW + (1.0 - rowhas) * orig

    dn0 = (((0,), (0,)), ((), ()))
    w = ohB * effp.astype(jnp.float32)
    hit = lax.dot_general(w, jnp.ones((B, 1), jnp.float32), dn0,
                          preferred_element_type=jnp.float32)
    n0 = lax.dot_general(w, posB.astype(jnp.float32), dn0,
                         preferred_element_type=jnp.float32)
    v1 = jnp.clip(ptrB1f + 1.0, 0.0, float(K))
    n1 = lax.dot_general(w, v1, dn0, preferred_element_type=jnp.float32)
    new0 = jnp.where(hit > 0, n0, ptrf[:, 0:1])
    new1 = jnp.where(hit > 0, n1, ptrf[:, 1:2])
    nptr_ref[...] = jnp.concatenate([new0, new1], axis=1).astype(jnp.int32)


def _route(concept, queue_ptr, g2, u, gB, q2d, tclsB):
    return pl.pallas_call(
        _route_body,
        out_shape=[
            jax.ShapeDtypeStruct((2 * B, 1), jnp.int32),
            jax.ShapeDtypeStruct((2 * B, 1), jnp.int32),
            jax.ShapeDtypeStruct((B, 1), jnp.int32),
            jax.ShapeDtypeStruct((C, 2), jnp.int32),
            jax.ShapeDtypeStruct((B, DIM), jnp.float32),
            jax.ShapeDtypeStruct((B, 1), jnp.int32),
        ],
    )(concept, queue_ptr.astype(jnp.float32), g2, u[:, None], gB, q2d, tclsB)


def _sc_queue(q2d, scvals, rb):
    info = plsc.get_sparse_core_info()
    nsub = info.num_subcores
    rows_per = R // nsub

    @functools.partial(
        pl.kernel,
        mesh=plsc.VectorSubcoreMesh(core_axis_name="c", subcore_axis_name="s"),
        out_type=jax.ShapeDtypeStruct((R, DIM), jnp.float32),
        scratch_types=[
            pltpu.VMEM((rows_per, DIM), jnp.float32),
            pltpu.VMEM((B,), jnp.int32),
            pltpu.VMEM((B, DIM), jnp.float32),
            pltpu.SemaphoreType.DMA,
        ],
    )
    def qk(q_ref, vals_ref, idx_ref, out_ref, rows_v, idx_v, vals_v, sem):
        cid = lax.axis_index("c")
        sid = lax.axis_index("s")

        @pl.when(cid == 0)
        def _():
            base = sid * rows_per
            pltpu.sync_copy(q_ref.at[pl.ds(base, rows_per)], rows_v)
            pltpu.sync_copy(rows_v, out_ref.at[pl.ds(base, rows_per)])
            plsc.subcore_barrier()

            @pl.when(sid == 0)
            def _():
                pltpu.sync_copy(idx_ref, idx_v)
                pltpu.sync_copy(vals_ref, vals_v)
                pltpu.async_copy(vals_v, out_ref.at[idx_v], sem).wait()

    return qk(q2d, scvals, rb)


def _bigcopy_body(ridx_ref, r2_ref, qg_ref, gs_ref, og_ref, q2a_ref, q2b_ref):
    og_ref[...] = qg_ref[...]
    base = pl.program_id(0) * K
    for i in range(B):
        r = ridx_ref[i]

        @pl.when((r >= base) & (r < base + K))
        def _(i=i, r=r):
            og_ref[0, :, pl.ds(r - base, 1), :] = gs_ref[:, i, :][:, None, :]

    for j in range(2 * B):
        rj = r2_ref[j]

        @pl.when((rj >= base) & (rj < base + K))
        def _(j=j, rj=rj):
            v = qg_ref[0, :, pl.ds(rj - base, 1), :].reshape(2 * N, DIM)
            q2a_ref[j] = v[:N]
            q2b_ref[j] = v[N:]


def _bigcopy(ridx, r2, qgT, gsT):
    grid_spec = pltpu.PrefetchScalarGridSpec(
        num_scalar_prefetch=2,
        grid=(C,),
        in_specs=[
            pl.BlockSpec((1, 2 * N, K, DIM), lambda g, s1, s2: (g, 0, 0, 0)),
            pl.BlockSpec((2 * N, B, DIM), lambda g, s1, s2: (0, 0, 0)),
        ],
        out_specs=[
            pl.BlockSpec((1, 2 * N, K, DIM), lambda g, s1, s2: (g, 0, 0, 0)),
            pl.BlockSpec((2 * B, N, DIM), lambda g, s1, s2: (0, 0, 0)),
            pl.BlockSpec((2 * B, N, DIM), lambda g, s1, s2: (0, 0, 0)),
        ],
    )
    return pl.pallas_call(
        _bigcopy_body,
        grid_spec=grid_spec,
        out_shape=[
            jax.ShapeDtypeStruct((C, 2 * N, K, DIM), jnp.float32),
            jax.ShapeDtypeStruct((2 * B, N, DIM), jnp.float32),
            jax.ShapeDtypeStruct((2 * B, N, DIM), jnp.float32),
        ],
    )(ridx, r2, qgT, gsT)


def _loss_body(q2a_ref, q2b_ref, tr_ref, tf_ref, sr_ref, sf_ref, vf_ref,
               cg_ref, out_ref, acc_ref):
    t = pl.program_id(0)

    @pl.when(t == 0)
    def _():
        acc_ref[0] = 0.0

    vm = (vf_ref[...] != 0.0)[:, :, None]
    ret2 = jnp.where(vm, q2a_ref[...], tr_ref[...])
    ret3 = jnp.where(vm, q2b_ref[...], tf_ref[...])

    x = (ret2 - cg_ref[...]) / TAU
    x = x - jnp.max(x, axis=-1, keepdims=True)
    e = jnp.exp(x)
    t_region = e / jnp.sum(e, axis=-1, keepdims=True)

    tn = ret3 / jnp.maximum(
        jnp.sqrt(jnp.sum(ret3 * ret3, axis=-1, keepdims=True)), 1e-12)
    sfv = sf_ref[...]
    sn = sfv / jnp.maximum(
        jnp.sqrt(jnp.sum(sfv * sfv, axis=-1, keepdims=True)), 1e-12)

    s = sr_ref[...] / TAU_STU
    ls = s - jnp.max(s, axis=-1, keepdims=True)
    ls = ls - jnp.log(jnp.sum(jnp.exp(ls), axis=-1, keepdims=True))

    total = 0.0
    for i in range(B):
        sim = lax.dot_general(sn[i], tn[i], (((1,), (1,)), ((), ())),
                              preferred_element_type=jnp.float32)
        mx = jnp.max(sim, axis=1, keepdims=True)
        ii = lax.broadcasted_iota(jnp.int32, sim.shape, 1)
        ind = jnp.min(jnp.where(sim == mx, ii, N), axis=1, keepdims=True)
        oh = (ind == lax.broadcasted_iota(jnp.int32, (N, N), 1)).astype(jnp.float32)
        t_indexed = jnp.dot(oh, t_region[i], preferred_element_type=jnp.float32)
        total = total + jnp.mean(jnp.sum(-t_indexed * ls[i], axis=-1))
    acc_ref[0] += 0.5 * total

    @pl.when(t == pl.num_programs(0) - 1)
    def _():
        out_ref[...] = jnp.full((1, 1), acc_ref[0] / (2.0 * B), jnp.float32)


def _loss(validf, q2a, q2b, t_region_out, t_fea, s_region_out, s_fea,
          center_grid):
    trow = pl.BlockSpec((B, N, DIM), lambda t: (t, 0, 0))
    srow = pl.BlockSpec((B, N, DIM), lambda t: (1 - t, 0, 0))
    grid_spec = pltpu.PrefetchScalarGridSpec(
        num_scalar_prefetch=0,
        grid=(2,),
        in_specs=[
            trow, trow, trow, trow, srow, srow,
            pl.BlockSpec((B, 1), lambda t: (t, 0)),
            pl.BlockSpec((1, DIM), lambda t: (0, 0)),
        ],
        out_specs=pl.BlockSpec((1, 1), lambda t: (0, 0)),
        scratch_shapes=[pltpu.SMEM((1,), jnp.float32)],
    )
    out = pl.pallas_call(
        _loss_body,
        grid_spec=grid_spec,
        out_shape=jax.ShapeDtypeStruct((1, 1), jnp.float32),
    )(q2a, q2b, t_region_out, t_fea, s_region_out, s_fea, validf, center_grid)
    return out[0, 0]


def kernel(t_cls_out, t_region_out, t_fea, s_cls_out, s_region_out, s_fea,
           concept, queue, queue_grid, queue_ptr, center, center_grid):
    key = jax.random.key(42)
    k1, k2, k3 = jax.random.split(key, 3)
    g2 = jax.random.gumbel(k1, (2 * B, C), jnp.float32)
    u = jax.random.uniform(k2, (2 * B,))
    gB = jax.random.gumbel(k3, (B, C), jnp.float32)

    q2d = queue.reshape(R, DIM)
    r2, valid, ridx, new_ptr, scvals, rb = _route(
        concept, queue_ptr, g2, u, gB, q2d, t_cls_out[:B])

    new_q = _sc_queue(q2d, scvals, rb.reshape(B))

    qgT = queue_grid.transpose(0, 2, 1, 3)
    gsT = jnp.concatenate([t_region_out[:B].transpose(1, 0, 2),
                           t_fea[:B].transpose(1, 0, 2)], axis=0)

    new_qgT, q2a, q2b = _bigcopy(ridx.reshape(B), r2.reshape(2 * B), qgT, gsT)

    total_loss = _loss(valid.astype(jnp.float32), q2a, q2b,
                       t_region_out, t_fea, s_region_out, s_fea, center_grid)

    return (total_loss,
            new_q.reshape(C, K, DIM),
            new_qgT.transpose(0, 2, 1, 3),
            new_ptr)
